# baseline (speedup 1.0000x reference)
import functools

import jax
import jax.numpy as jnp
from jax import lax
from jax.experimental import pallas as pl
from jax.experimental.pallas import tpu as pltpu
from jax.experimental.pallas import tpu_sc as plsc

L = 50
B = 16384
D = 128
CB = 2
ROWS = CB * L
NSLOT = 4

NC = 2
NS = 16
NW = NC * NS
BPW = B // NW
CPW = BPW // CB
LANE = 16
DV = D // LANE

EPI = NSLOT * CB
IPF = LANE // EPI


def _pool_dot(rows_n, rows_d, s, e):
  def jbody(j, acc):
    base = (s * CB + e) * L + j
    new = []
    for d in range(DV):
      sl = pl.ds(d * LANE, LANE)
      new.append(acc[d] + rows_n[base, sl])
    for d in range(DV):
      sl = pl.ds(d * LANE, LANE)
      new.append(acc[DV + d] + rows_d[base, sl])
    return tuple(new)

  init = tuple(jnp.zeros((LANE,), jnp.float32) for _ in range(2 * DV))
  acc = lax.fori_loop(0, L, jbody, init, unroll=5)
  p = acc[0] * acc[DV]
  for d in range(1, DV):
    p = p + acc[d] * acc[DV + d]
  lanes = lax.iota(jnp.int32, LANE)
  for k in (1, 2, 4, 8):
    p = p + p.at[lanes ^ k].get(mode="promise_in_bounds")
  return p


_mesh = plsc.VectorSubcoreMesh(core_axis_name="c", subcore_axis_name="s")


@functools.partial(
    pl.kernel,
    out_type=jax.ShapeDtypeStruct((B,), jnp.float32),
    mesh=_mesh,
    scratch_types=[
        pltpu.VMEM((2 * NSLOT, ROWS), jnp.int32),
        pltpu.VMEM((2 * NSLOT, ROWS), jnp.int32),
        pltpu.VMEM((NSLOT * ROWS, D), jnp.float32),
        pltpu.VMEM((NSLOT * ROWS, D), jnp.float32),
        pltpu.VMEM((BPW,), jnp.float32),
        [pltpu.SemaphoreType.DMA] * NSLOT,
        [pltpu.SemaphoreType.DMA] * NSLOT,
        [pltpu.SemaphoreType.DMA] * (2 * NSLOT),
        [pltpu.SemaphoreType.DMA] * (2 * NSLOT),
    ],
)
def _sc_fwd(wn_idx, wd_idx, wn_tab, wd_tab, out_hbm,
            idx_n, idx_d, rows_n, rows_d, out_v,
            gsem_n, gsem_d, isem_n, isem_d):
  wid = lax.axis_index("s") * NC + lax.axis_index("c")

  def start_idx(c, islot):
    g = wid * CPW + c
    pltpu.async_copy(wn_idx.at[g], idx_n.at[islot], isem_n[islot])
    pltpu.async_copy(wd_idx.at[g], idx_d.at[islot], isem_d[islot])

  def wait_idx(c, islot):
    g = wid * CPW + c
    pltpu.make_async_copy(wn_idx.at[g], idx_n.at[islot], isem_n[islot]).wait()
    pltpu.make_async_copy(wd_idx.at[g], idx_d.at[islot], isem_d[islot]).wait()

  def start_gather(slot, islot):
    dst = pl.ds(slot * ROWS, ROWS)
    pltpu.async_copy(wn_tab.at[idx_n.at[islot]], rows_n.at[dst], gsem_n[slot])
    pltpu.async_copy(wd_tab.at[idx_d.at[islot]], rows_d.at[dst], gsem_d[slot])

  def wait_gather(slot, islot):
    dst = pl.ds(slot * ROWS, ROWS)
    pltpu.make_async_copy(
        wn_tab.at[idx_n.at[islot]], rows_n.at[dst], gsem_n[slot]).wait()
    pltpu.make_async_copy(
        wd_tab.at[idx_d.at[islot]], rows_d.at[dst], gsem_d[slot]).wait()

  lanes = lax.iota(jnp.int32, LANE)

  for k in range(2 * NSLOT):
    start_idx(k, k)
  for k in range(NSLOT):
    wait_idx(k, k)
    start_gather(k, k)

  def chunk_body(i, vec):
    for s2 in range(2 * NSLOT):
      c = 2 * NSLOT * i + s2
      slot = s2 % NSLOT
      wait_gather(slot, s2)

      for e in range(CB):
        p = _pool_dot(rows_n, rows_d, slot, e)
        vec = jnp.where(lanes == s2 * CB + e, p, vec)

      nxt = c + NSLOT

      @pl.when(nxt < CPW)
      def _():
        wait_idx(nxt, (s2 + NSLOT) % (2 * NSLOT))
        start_gather(slot, (s2 + NSLOT) % (2 * NSLOT))

      nx8 = c + 2 * NSLOT

      @pl.when(nx8 < CPW)
      def _():
        start_idx(nx8, s2)

    out_v[pl.ds(i * LANE, LANE)] = vec
    return vec

  lax.fori_loop(0, CPW // (2 * NSLOT), chunk_body,
                jnp.zeros((LANE,), jnp.float32))

  def sig_body(k, carry):
    sl = pl.ds(k * LANE, LANE)
    v = out_v[sl]
    out_v[sl] = 1.0 / (1.0 + jnp.exp(-v))
    return carry

  lax.fori_loop(0, BPW // LANE, sig_body, 0)

  pltpu.sync_copy(out_v, out_hbm.at[pl.ds(wid * BPW, BPW)])


@jax.jit
def kernel(wn_path, wd_path, wn_table, wd_table):
  wn_idx = wn_path.T.reshape(B // CB, ROWS)
  wd_idx = wd_path.T.reshape(B // CB, ROWS)
  out = _sc_fwd(wn_idx, wd_idx, wn_table, wd_table)
  return out.reshape(B, 1, 1)


# speedup vs baseline: 1.0200x; 1.0200x over previous
import functools

import jax
import jax.numpy as jnp
from jax import lax
from jax.experimental import pallas as pl
from jax.experimental.pallas import tpu as pltpu
from jax.experimental.pallas import tpu_sc as plsc

L = 50
B = 16384
D = 128
CB = 2
ROWS = CB * L
NSLOT = 4

NC = 2
NS = 16
NW = NC * NS
BPW = B // NW
CPW = BPW // CB
LANE = 16
DV = D // LANE

EPI = NSLOT * CB
IPF = LANE // EPI


def _pool_dot(rows_n, rows_d, s, e):
  def jbody(j, acc):
    base = (s * CB + e) * L + j
    new = []
    for d in range(DV):
      sl = pl.ds(d * LANE, LANE)
      new.append(acc[d] + rows_n[base, sl])
    for d in range(DV):
      sl = pl.ds(d * LANE, LANE)
      new.append(acc[DV + d] + rows_d[base, sl])
    return tuple(new)

  init = tuple(jnp.zeros((LANE,), jnp.float32) for _ in range(2 * DV))
  acc = lax.fori_loop(0, L, jbody, init, unroll=5)
  p = acc[0] * acc[DV]
  for d in range(1, DV):
    p = p + acc[d] * acc[DV + d]
  lanes = lax.iota(jnp.int32, LANE)
  for k in (1, 2, 4, 8):
    p = p + p.at[lanes ^ k].get(mode="promise_in_bounds")
  return p


_mesh = plsc.VectorSubcoreMesh(core_axis_name="c", subcore_axis_name="s")


@functools.partial(
    pl.kernel,
    out_type=jax.ShapeDtypeStruct((B,), jnp.float32),
    mesh=_mesh,
    scratch_types=[
        pltpu.VMEM((NSLOT, ROWS), jnp.int32),
        pltpu.VMEM((NSLOT, ROWS), jnp.int32),
        pltpu.VMEM((NSLOT * ROWS, D), jnp.float32),
        pltpu.VMEM((NSLOT * ROWS, D), jnp.float32),
        pltpu.VMEM((BPW,), jnp.float32),
        [pltpu.SemaphoreType.DMA] * NSLOT,
        [pltpu.SemaphoreType.DMA] * NSLOT,
        [pltpu.SemaphoreType.DMA] * NSLOT,
        [pltpu.SemaphoreType.DMA] * NSLOT,
    ],
)
def _sc_fwd(wn_idx, wd_idx, wn_tab, wd_tab, out_hbm,
            idx_n, idx_d, rows_n, rows_d, out_v,
            gsem_n, gsem_d, isem_n, isem_d):
  wid = lax.axis_index("s") * NC + lax.axis_index("c")

  def start_idx(c, slot):
    g = wid * CPW + c
    pltpu.async_copy(wn_idx.at[g], idx_n.at[slot], isem_n[slot])
    pltpu.async_copy(wd_idx.at[g], idx_d.at[slot], isem_d[slot])

  def wait_idx(c, slot):
    g = wid * CPW + c
    pltpu.make_async_copy(wn_idx.at[g], idx_n.at[slot], isem_n[slot]).wait()
    pltpu.make_async_copy(wd_idx.at[g], idx_d.at[slot], isem_d[slot]).wait()

  def start_gather(slot):
    dst = pl.ds(slot * ROWS, ROWS)
    pltpu.async_copy(wn_tab.at[idx_n.at[slot]], rows_n.at[dst], gsem_n[slot])
    pltpu.async_copy(wd_tab.at[idx_d.at[slot]], rows_d.at[dst], gsem_d[slot])

  def wait_gather(slot):
    dst = pl.ds(slot * ROWS, ROWS)
    pltpu.make_async_copy(
        wn_tab.at[idx_n.at[slot]], rows_n.at[dst], gsem_n[slot]).wait()
    pltpu.make_async_copy(
        wd_tab.at[idx_d.at[slot]], rows_d.at[dst], gsem_d[slot]).wait()

  lanes = lax.iota(jnp.int32, LANE)

  for k in range(NSLOT):
    start_idx(k, k)
  for k in range(NSLOT - 1):
    wait_idx(k, k)
    start_gather(k)

  def chunk_body(i, vec):
    ph = lax.rem(i, IPF) * EPI
    for s in range(NSLOT):
      c = NSLOT * i + s
      wait_gather(s)
      nxt = c + NSLOT - 1

      @pl.when(nxt < CPW)
      def _():
        wait_idx(nxt, (s + NSLOT - 1) % NSLOT)
        start_gather((s + NSLOT - 1) % NSLOT)

      nx4 = c + NSLOT

      @pl.when(nx4 < CPW)
      def _():
        start_idx(nx4, s)

      for e in range(CB):
        p = _pool_dot(rows_n, rows_d, s, e)
        vec = jnp.where(lanes == ph + s * CB + e, p, vec)

    @pl.when(lax.rem(i, IPF) == IPF - 1)
    def _():
      out_v[pl.ds((i // IPF) * LANE, LANE)] = vec

    return vec

  lax.fori_loop(0, CPW // NSLOT, chunk_body, jnp.zeros((LANE,), jnp.float32))

  def sig_body(k, carry):
    sl = pl.ds(k * LANE, LANE)
    v = out_v[sl]
    out_v[sl] = 1.0 / (1.0 + jnp.exp(-v))
    return carry

  lax.fori_loop(0, BPW // LANE, sig_body, 0)

  pltpu.sync_copy(out_v, out_hbm.at[pl.ds(wid * BPW, BPW)])


@jax.jit
def kernel(wn_path, wd_path, wn_table, wd_table):
  wn_idx = wn_path.T.reshape(B // CB, ROWS)
  wd_idx = wd_path.T.reshape(B // CB, ROWS)
  out = _sc_fwd(wn_idx, wd_idx, wn_table, wd_table)
  return out.reshape(B, 1, 1)
